# Initial kernel scaffold; baseline (speedup 1.0000x reference)
import jax
import jax.numpy as jnp
from jax.experimental import pallas as pl


def kernel(x, pe_weight):
    raise NotImplementedError("write your pallas kernel here")


# speedup vs baseline: 1.4919x; 1.4919x over previous
import jax
import jax.numpy as jnp
from jax.experimental import pallas as pl
from jax.experimental.pallas import tpu as pltpu

SEQ_BLOCK = 512


def _add_body(x_ref, pe_ref, o_ref):
    o_ref[...] = x_ref[...] + pe_ref[...][None, :, :]


def kernel(x, pe_weight):
    batch, seq_len, d_model = x.shape
    pe = pe_weight[:seq_len]
    num_seq_blocks = seq_len // SEQ_BLOCK

    grid = (num_seq_blocks, batch)
    return pl.pallas_call(
        _add_body,
        grid=grid,
        in_specs=[
            pl.BlockSpec((1, SEQ_BLOCK, d_model), lambda i, j: (j, i, 0)),
            pl.BlockSpec((SEQ_BLOCK, d_model), lambda i, j: (i, 0)),
        ],
        out_specs=pl.BlockSpec((1, SEQ_BLOCK, d_model), lambda i, j: (j, i, 0)),
        out_shape=jax.ShapeDtypeStruct(x.shape, x.dtype),
        compiler_params=pltpu.CompilerParams(
            dimension_semantics=("arbitrary", "arbitrary"),
        ),
    )(x, pe)
